# baseline (speedup 1.0000x reference)
import jax
import jax.numpy as jnp
from jax import lax
from jax.experimental import pallas as pl
from jax.experimental.pallas import tpu as pltpu
from jax.experimental.pallas import tpu_sc as plsc

_ALPHA = 0.1
_BETA = 0.9
_NUMK = 4

_NC = 2
_NS = 16
_NW = _NC * _NS
_CH = 125
_WIN = 5120
_ACC_R = 5248


def _remap_windows(didx_v, dloc_v, nchunks):
    def remap_row(j, carry):
        for w in range(2):
            base = w * _WIN
            for o in (0, 16, 32, 48, 64, 80, 96, _CH - 16):
                v = didx_v.at[j][pl.ds(o, 16)]
                loc = v - base
                ok = (loc >= 0) & (loc < _WIN)
                dloc_v.at[w].at[j][pl.ds(o, 16)] = jnp.where(ok, loc, _WIN)
        return carry

    lax.fori_loop(0, nchunks, remap_row, 0)


def _remap_one_window(didx_v, dw_v, nchunks, base):
    def remap_row(j, carry):
        for o in (0, 16, 32, 48, 64, 80, 96, _CH - 16):
            v = didx_v.at[j][pl.ds(o, 16)]
            loc = v - base
            ok = (loc >= 0) & (loc < _WIN)
            dw_v.at[j][pl.ds(o, 16)] = jnp.where(ok, loc, _WIN)
        return carry

    lax.fori_loop(0, nchunks, remap_row, 0)


def _deg_body(dst3_hbm, zerosd_hbm, onesd_hbm, degp_hbm,
              didx_v, dloc_v, ones_v, acc_sh, *ssems):
    nchunks = dst3_hbm.shape[1]
    rows_per_s = _ACC_R // _NS
    c = lax.axis_index("c")
    s = lax.axis_index("s")
    wid = s * _NC + c
    row0 = s * rows_per_s
    pltpu.sync_copy(dst3_hbm.at[wid], didx_v)
    pltpu.sync_copy(onesd_hbm, ones_v)
    _remap_windows(didx_v, dloc_v, nchunks)

    nbuf = len(ssems)
    for w in range(2):
        dw = dloc_v.at[w]
        pltpu.sync_copy(
            zerosd_hbm.at[pl.ds(row0, rows_per_s)],
            acc_sh.at[pl.ds(row0, rows_per_s)],
        )
        plsc.subcore_barrier()

        def quad(p, carry):
            j = p * nbuf
            for b in range(nbuf):
                @pl.when(j + b >= nbuf)
                def _():
                    pltpu.make_async_copy(
                        ones_v, acc_sh.at[dw.at[j + b - nbuf]], ssems[b]
                    ).wait()
                pltpu.async_copy(ones_v, acc_sh.at[dw.at[j + b]], ssems[b],
                                 add=True)
            return carry

        lax.fori_loop(0, nchunks // nbuf, quad, 0)
        for b in range(nbuf):
            pltpu.make_async_copy(
                ones_v, acc_sh.at[dw.at[nchunks - nbuf + b]], ssems[b]
            ).wait()
        plsc.subcore_barrier()
        pltpu.sync_copy(
            acc_sh.at[pl.ds(row0, rows_per_s)],
            degp_hbm.at[c].at[w].at[pl.ds(row0, rows_per_s)],
        )


def _round_body(g_hbm, src3_hbm, dst3_hbm, zerosd_hbm, accp_hbm,
                sidx_v, didx_v, dw_v, rows_v, acc_sh, *sems):
    nchunks = src3_hbm.shape[1]
    nbuf = rows_v.shape[0]
    rows_per_s = _ACC_R // _NS
    c = lax.axis_index("c")
    s = lax.axis_index("s")
    wid = s * _NC + c
    row0 = s * rows_per_s
    pltpu.sync_copy(src3_hbm.at[wid], sidx_v)
    pltpu.sync_copy(dst3_hbm.at[wid], didx_v)

    def window(w, wcarry):
        _remap_one_window(didx_v, dw_v, nchunks, w * _WIN)
        dw = dw_v
        pltpu.sync_copy(
            zerosd_hbm.at[pl.ds(row0, rows_per_s)],
            acc_sh.at[pl.ds(row0, rows_per_s)],
        )
        plsc.subcore_barrier()

        def step(p, carry):
            j = p * nbuf
            @pl.when(j >= nbuf)
            def _():
                for b in range(nbuf):
                    pltpu.make_async_copy(
                        g_hbm.at[sidx_v.at[j + b - nbuf]], rows_v.at[b],
                        sems[b]
                    ).wait()
                    pltpu.async_copy(
                        rows_v.at[b], acc_sh.at[dw.at[j + b - nbuf]],
                        sems[b], add=True)
            for b in range(nbuf):
                @pl.when(j + b < nchunks)
                def _():
                    @pl.when(j >= nbuf)
                    def _():
                        pltpu.make_async_copy(
                            rows_v.at[b], acc_sh.at[dw.at[j + b - nbuf]],
                            sems[b]
                        ).wait()
                    pltpu.async_copy(
                        g_hbm.at[sidx_v.at[j + b]], rows_v.at[b], sems[b]
                    )
            return carry

        lax.fori_loop(0, nchunks // nbuf + 1, step, 0)
        for b in range(nbuf):
            pltpu.make_async_copy(
                rows_v.at[b], acc_sh.at[dw.at[nchunks - nbuf + b]], sems[b]
            ).wait()
        plsc.subcore_barrier()
        pltpu.sync_copy(
            acc_sh.at[pl.ds(row0, rows_per_s)],
            accp_hbm.at[c].at[w].at[pl.ds(row0, rows_per_s)],
        )
        return wcarry

    lax.fori_loop(0, 2, window, 0)


def _sc_deg(dst3, zerosd, onesd):
    D = zerosd.shape[1]
    nchunks = dst3.shape[1]
    mesh = plsc.VectorSubcoreMesh(core_axis_name="c", subcore_axis_name="s")
    return pl.kernel(
        _deg_body,
        out_type=jax.ShapeDtypeStruct((_NC, 2, _ACC_R, D), jnp.float32),
        mesh=mesh,
        scratch_types=[
            pltpu.VMEM((nchunks, _CH), jnp.int32),
            pltpu.VMEM((2, nchunks, _CH), jnp.int32),
            pltpu.VMEM((_CH, D), jnp.float32),
            pltpu.VMEM_SHARED((_ACC_R, D), jnp.float32),
            pltpu.SemaphoreType.DMA,
            pltpu.SemaphoreType.DMA,
            pltpu.SemaphoreType.DMA,
            pltpu.SemaphoreType.DMA,
        ],
    )(dst3, zerosd, onesd)


def _sc_round(g, src3, dst3, zerosd):
    N, D = g.shape
    nchunks = src3.shape[1]
    mesh = plsc.VectorSubcoreMesh(core_axis_name="c", subcore_axis_name="s")
    return pl.kernel(
        _round_body,
        out_type=jax.ShapeDtypeStruct((_NC, 2, _ACC_R, D), jnp.float32),
        mesh=mesh,
        scratch_types=[
            pltpu.VMEM((nchunks, _CH), jnp.int32),
            pltpu.VMEM((nchunks, _CH), jnp.int32),
            pltpu.VMEM((nchunks, _CH), jnp.int32),
            pltpu.VMEM((2, _CH, D), jnp.float32),
            pltpu.VMEM_SHARED((_ACC_R, D), jnp.float32),
        ] + [pltpu.SemaphoreType.DMA] * 4,
    )(g, src3, dst3, zerosd)


def _tc_dinv_g(degp, x):
    N, D = x.shape
    blk = 1024
    bpw = _WIN // blk

    def body(degp_ref, x_ref, dinv_ref, g_ref):
        deg = degp_ref[0, 0, :, :8] + degp_ref[1, 0, :, :8]
        dinv = jnp.where(deg > 0.0, lax.rsqrt(jnp.maximum(deg, 1.0)), 0.0)
        dinv_ref[...] = dinv
        g_ref[...] = x_ref[...] * dinv[:, :1]

    return pl.pallas_call(
        body,
        grid=(N // blk,),
        in_specs=[
            pl.BlockSpec((2, 1, blk, D), lambda i: (0, i // bpw, i % bpw, 0)),
            pl.BlockSpec((blk, D), lambda i: (i, 0)),
        ],
        out_specs=[
            pl.BlockSpec((blk, 8), lambda i: (i, 0)),
            pl.BlockSpec((blk, D), lambda i: (i, 0)),
        ],
        out_shape=[
            jax.ShapeDtypeStruct((N, 8), jnp.float32),
            jax.ShapeDtypeStruct((N, D), jnp.float32),
        ],
    )(degp, x)


def _tc_combine(accp, x, dinv8, scale):
    N, D = x.shape
    blk = 1024
    bpw = _WIN // blk

    def body(scale_ref, accp_ref, x_ref, dinv_ref, g_ref, h_ref):
        acc = accp_ref[0, 0] + accp_ref[1, 0]
        dv = dinv_ref[:, :1]
        hs = (_BETA * dv) * acc + _ALPHA * x_ref[...]
        g_ref[...] = hs * dv
        h_ref[...] = hs * scale_ref[0]

    return pl.pallas_call(
        body,
        grid=(N // blk,),
        in_specs=[
            pl.BlockSpec(memory_space=pltpu.SMEM),
            pl.BlockSpec((2, 1, blk, D), lambda i: (0, i // bpw, i % bpw, 0)),
            pl.BlockSpec((blk, D), lambda i: (i, 0)),
            pl.BlockSpec((blk, 8), lambda i: (i, 0)),
        ],
        out_specs=[
            pl.BlockSpec((blk, D), lambda i: (i, 0)),
            pl.BlockSpec((blk, D), lambda i: (i, 0)),
        ],
        out_shape=[
            jax.ShapeDtypeStruct((N, D), jnp.float32),
            jax.ShapeDtypeStruct((N, D), jnp.float32),
        ],
    )(scale, accp, x, dinv8)


@jax.jit
def kernel(x, edge_index):
    N, D = x.shape
    E = edge_index.shape[1]
    npad = -(-N // 1280) * 1280
    xp = jnp.pad(x, ((0, npad - N), (0, 0)))
    nchunks = E // _NW // _CH
    src3 = edge_index[0].reshape(_NW, nchunks, _CH)
    dst3 = edge_index[1].reshape(_NW, nchunks, _CH)
    onesd = jnp.ones((_CH, D), jnp.float32)
    zerosd = jnp.zeros((npad, D), jnp.float32)

    gamma = _BETA ** _NUMK + _ALPHA * sum(_BETA ** i for i in range(_NUMK))

    degp = _sc_deg(dst3, zerosd, onesd)
    dinv8, g = _tc_dinv_g(degp, xp)

    def round_step(k, carry):
        g, _ = carry
        accp = _sc_round(g, src3, dst3, zerosd)
        scale = jnp.where(k == _NUMK - 1, 1.0 / gamma, 1.0).reshape(1)
        return tuple(_tc_combine(accp, xp, dinv8, scale))

    _, out = lax.fori_loop(0, _NUMK, round_step, (g, g))
    return out[:N]


# speedup vs baseline: 1.1284x; 1.1284x over previous
import jax
import jax.numpy as jnp
from jax import lax
from jax.experimental import pallas as pl
from jax.experimental.pallas import tpu as pltpu
from jax.experimental.pallas import tpu_sc as plsc

_ALPHA = 0.1
_BETA = 0.9
_NUMK = 4

_NC = 2
_NS = 16
_NW = _NC * _NS
_CH = 125
_WIN = 5120
_ACC_R = 5248


def _remap_windows(didx_v, dloc_v, nchunks):
    def remap_row(j, carry):
        for w in range(2):
            base = w * _WIN
            for o in (0, 16, 32, 48, 64, 80, 96, _CH - 16):
                v = didx_v.at[j][pl.ds(o, 16)]
                loc = v - base
                ok = (loc >= 0) & (loc < _WIN)
                dloc_v.at[w].at[j][pl.ds(o, 16)] = jnp.where(ok, loc, _WIN)
        return carry

    lax.fori_loop(0, nchunks, remap_row, 0)


def _remap_one_window(didx_v, dw_v, nchunks, base):
    def remap_row(j, carry):
        for o in (0, 16, 32, 48, 64, 80, 96, _CH - 16):
            v = didx_v.at[j][pl.ds(o, 16)]
            loc = v - base
            ok = (loc >= 0) & (loc < _WIN)
            dw_v.at[j][pl.ds(o, 16)] = jnp.where(ok, loc, _WIN)
        return carry

    lax.fori_loop(0, nchunks, remap_row, 0)


def _deg_body(dst3_hbm, zerosd_hbm, onesd_hbm, degp_hbm,
              didx_v, dloc_v, ones_v, acc_sh, *ssems):
    nchunks = dst3_hbm.shape[1]
    rows_per_s = _ACC_R // _NS
    c = lax.axis_index("c")
    s = lax.axis_index("s")
    wid = s * _NC + c
    row0 = s * rows_per_s
    pltpu.sync_copy(dst3_hbm.at[wid], didx_v)
    pltpu.sync_copy(onesd_hbm, ones_v)
    _remap_windows(didx_v, dloc_v, nchunks)

    nbuf = len(ssems)
    for w in range(2):
        dw = dloc_v.at[w]
        pltpu.sync_copy(
            zerosd_hbm.at[pl.ds(row0, rows_per_s)],
            acc_sh.at[pl.ds(row0, rows_per_s)],
        )
        plsc.subcore_barrier()

        def quad(p, carry):
            j = p * nbuf
            for b in range(nbuf):
                @pl.when(j + b >= nbuf)
                def _():
                    pltpu.make_async_copy(
                        ones_v, acc_sh.at[dw.at[j + b - nbuf]], ssems[b]
                    ).wait()
                pltpu.async_copy(ones_v, acc_sh.at[dw.at[j + b]], ssems[b],
                                 add=True)
            return carry

        lax.fori_loop(0, nchunks // nbuf, quad, 0)
        for b in range(nbuf):
            pltpu.make_async_copy(
                ones_v, acc_sh.at[dw.at[nchunks - nbuf + b]], ssems[b]
            ).wait()
        plsc.subcore_barrier()
        pltpu.sync_copy(
            acc_sh.at[pl.ds(row0, rows_per_s)],
            degp_hbm.at[c].at[w].at[pl.ds(row0, rows_per_s)],
        )


def _round_body(g_hbm, src3_hbm, dst3_hbm, zerosd_hbm, accp_hbm,
                sidx_v, didx_v, dw_v, rows_v, acc_sh, *sems):
    nchunks = src3_hbm.shape[1]
    nbuf = rows_v.shape[0]
    rows_per_s = _ACC_R // _NS
    c = lax.axis_index("c")
    s = lax.axis_index("s")
    wid = s * _NC + c
    row0 = s * rows_per_s
    pltpu.sync_copy(src3_hbm.at[wid], sidx_v)
    pltpu.sync_copy(dst3_hbm.at[wid], didx_v)

    def window(w, wcarry):
        _remap_one_window(didx_v, dw_v, nchunks, w * _WIN)
        dw = dw_v
        pltpu.sync_copy(
            zerosd_hbm.at[pl.ds(row0, rows_per_s)],
            acc_sh.at[pl.ds(row0, rows_per_s)],
        )
        plsc.subcore_barrier()

        for b in range(nbuf):
            pltpu.async_copy(g_hbm.at[sidx_v.at[b]], rows_v.at[b], sems[b])

        def step(p, carry):
            j = p * nbuf
            for b in range(nbuf):
                @pl.when(j + b < nchunks)
                def _():
                    pltpu.make_async_copy(
                        g_hbm.at[sidx_v.at[j + b]], rows_v.at[b], sems[b]
                    ).wait()
                    pltpu.sync_copy(rows_v.at[b], acc_sh.at[dw.at[j + b]],
                                    add=True)

                    @pl.when(j + b + nbuf < nchunks)
                    def _():
                        pltpu.async_copy(
                            g_hbm.at[sidx_v.at[j + b + nbuf]], rows_v.at[b],
                            sems[b]
                        )
            return carry

        lax.fori_loop(0, -(-nchunks // nbuf), step, 0)
        plsc.subcore_barrier()
        pltpu.sync_copy(
            acc_sh.at[pl.ds(row0, rows_per_s)],
            accp_hbm.at[c].at[w].at[pl.ds(row0, rows_per_s)],
        )
        return wcarry

    lax.fori_loop(0, 2, window, 0)


def _sc_deg(dst3, zerosd, onesd):
    D = zerosd.shape[1]
    nchunks = dst3.shape[1]
    mesh = plsc.VectorSubcoreMesh(core_axis_name="c", subcore_axis_name="s")
    return pl.kernel(
        _deg_body,
        out_type=jax.ShapeDtypeStruct((_NC, 2, _ACC_R, D), jnp.float32),
        mesh=mesh,
        scratch_types=[
            pltpu.VMEM((nchunks, _CH), jnp.int32),
            pltpu.VMEM((2, nchunks, _CH), jnp.int32),
            pltpu.VMEM((_CH, D), jnp.float32),
            pltpu.VMEM_SHARED((_ACC_R, D), jnp.float32),
            pltpu.SemaphoreType.DMA,
            pltpu.SemaphoreType.DMA,
            pltpu.SemaphoreType.DMA,
            pltpu.SemaphoreType.DMA,
        ],
    )(dst3, zerosd, onesd)


def _sc_round(g, src3, dst3, zerosd):
    N, D = g.shape
    nchunks = src3.shape[1]
    mesh = plsc.VectorSubcoreMesh(core_axis_name="c", subcore_axis_name="s")
    return pl.kernel(
        _round_body,
        out_type=jax.ShapeDtypeStruct((_NC, 2, _ACC_R, D), jnp.float32),
        mesh=mesh,
        scratch_types=[
            pltpu.VMEM((nchunks, _CH), jnp.int32),
            pltpu.VMEM((nchunks, _CH), jnp.int32),
            pltpu.VMEM((nchunks, _CH), jnp.int32),
            pltpu.VMEM((3, _CH, D), jnp.float32),
            pltpu.VMEM_SHARED((_ACC_R, D), jnp.float32),
        ] + [pltpu.SemaphoreType.DMA] * 3,
    )(g, src3, dst3, zerosd)


def _tc_dinv_g(degp, x):
    N, D = x.shape
    blk = 1024
    bpw = _WIN // blk

    def body(degp_ref, x_ref, dinv_ref, g_ref):
        deg = degp_ref[0, 0, :, :8] + degp_ref[1, 0, :, :8]
        dinv = jnp.where(deg > 0.0, lax.rsqrt(jnp.maximum(deg, 1.0)), 0.0)
        dinv_ref[...] = dinv
        g_ref[...] = x_ref[...] * dinv[:, :1]

    return pl.pallas_call(
        body,
        grid=(N // blk,),
        in_specs=[
            pl.BlockSpec((2, 1, blk, D), lambda i: (0, i // bpw, i % bpw, 0)),
            pl.BlockSpec((blk, D), lambda i: (i, 0)),
        ],
        out_specs=[
            pl.BlockSpec((blk, 8), lambda i: (i, 0)),
            pl.BlockSpec((blk, D), lambda i: (i, 0)),
        ],
        out_shape=[
            jax.ShapeDtypeStruct((N, 8), jnp.float32),
            jax.ShapeDtypeStruct((N, D), jnp.float32),
        ],
    )(degp, x)


def _tc_combine(accp, x, dinv8, scale):
    N, D = x.shape
    blk = 1024
    bpw = _WIN // blk

    def body(scale_ref, accp_ref, x_ref, dinv_ref, g_ref, h_ref):
        acc = accp_ref[0, 0] + accp_ref[1, 0]
        dv = dinv_ref[:, :1]
        hs = (_BETA * dv) * acc + _ALPHA * x_ref[...]
        g_ref[...] = hs * dv
        h_ref[...] = hs * scale_ref[0]

    return pl.pallas_call(
        body,
        grid=(N // blk,),
        in_specs=[
            pl.BlockSpec(memory_space=pltpu.SMEM),
            pl.BlockSpec((2, 1, blk, D), lambda i: (0, i // bpw, i % bpw, 0)),
            pl.BlockSpec((blk, D), lambda i: (i, 0)),
            pl.BlockSpec((blk, 8), lambda i: (i, 0)),
        ],
        out_specs=[
            pl.BlockSpec((blk, D), lambda i: (i, 0)),
            pl.BlockSpec((blk, D), lambda i: (i, 0)),
        ],
        out_shape=[
            jax.ShapeDtypeStruct((N, D), jnp.float32),
            jax.ShapeDtypeStruct((N, D), jnp.float32),
        ],
    )(scale, accp, x, dinv8)


@jax.jit
def kernel(x, edge_index):
    N, D = x.shape
    E = edge_index.shape[1]
    npad = -(-N // 1280) * 1280
    xp = jnp.pad(x, ((0, npad - N), (0, 0)))
    nchunks = E // _NW // _CH
    src3 = edge_index[0].reshape(_NW, nchunks, _CH)
    dst3 = edge_index[1].reshape(_NW, nchunks, _CH)
    onesd = jnp.ones((_CH, D), jnp.float32)
    zerosd = jnp.zeros((npad, D), jnp.float32)

    gamma = _BETA ** _NUMK + _ALPHA * sum(_BETA ** i for i in range(_NUMK))

    degp = _sc_deg(dst3, zerosd, onesd)
    dinv8, g = _tc_dinv_g(degp, xp)

    def round_step(k, carry):
        g, _ = carry
        accp = _sc_round(g, src3, dst3, zerosd)
        scale = jnp.where(k == _NUMK - 1, 1.0 / gamma, 1.0).reshape(1)
        return tuple(_tc_combine(accp, xp, dinv8, scale))

    _, out = lax.fori_loop(0, _NUMK, round_step, (g, g))
    return out[:N]
